# baseline (speedup 1.0000x reference)
import jax
import jax.numpy as jnp
import numpy as np
from jax.experimental import pallas as pl
from jax.experimental.pallas import tpu as pltpu

B = 4096
D = 384
H = 64
R = 1024
NR = 8
AUX = 0.05
TB = 1024
NCHUNK = 8
CW = R // NCHUNK

_PREC = jax.lax.Precision.HIGHEST

def _threefry_uniform(seed, n):
    old = np.seterr(over="ignore")
    k0, k1 = np.uint32(0), np.uint32(seed)
    ks2 = np.uint32(0x1BD11BDA) ^ k0 ^ k1
    rot = [[13, 15, 26, 6], [17, 29, 16, 24]]
    x0 = np.zeros(n, np.uint32) + k0
    x1 = np.arange(n, dtype=np.uint32) + k1
    ks = [k0, k1, ks2]
    for i in range(5):
        for r in rot[i % 2]:
            x0 = x0 + x1
            x1 = (x1 << np.uint32(r)) | (x1 >> np.uint32(32 - r))
            x1 = x1 ^ x0
        x0 = x0 + ks[(i + 1) % 3]
        x1 = x1 + ks[(i + 2) % 3] + np.uint32(i + 1)
    bits = x0 ^ x1
    np.seterr(**old)
    bits = (bits >> np.uint32(9)) | np.uint32(0x3F800000)
    return bits.view(np.float32) - np.float32(1.0)


_RND = _threefry_uniform(42, B).reshape(B, 1)


def _dot(a, b, dims):
    return jax.lax.dot_general(a, b, (dims, ((), ())),
                               preferred_element_type=jnp.float32,
                               precision=_PREC)


def _split(a):
    hi = a.astype(jnp.bfloat16)
    lo = (a - hi.astype(jnp.float32)).astype(jnp.bfloat16)
    return hi, lo


def _dot1(a, b, dims):
    return jax.lax.dot_general(a, b, (dims, ((), ())),
                               preferred_element_type=jnp.float32,
                               precision=jax.lax.Precision.DEFAULT)


def _dot3s(a_hi, a_lo, b_hi, b_lo, dims):
    return (_dot1(a_hi, b_hi, dims) + _dot1(a_lo, b_hi, dims)
            + _dot1(a_hi, b_lo, dims))


def _dot2i(a, b_exact, dims):
    ah, al = _split(a)
    bh = b_exact.astype(jnp.bfloat16)
    return _dot1(ah, bh, dims) + _dot1(al, bh, dims)


def _dot2i_l(a_exact, b, dims):
    bh, bl = _split(b)
    ah = a_exact.astype(jnp.bfloat16)
    return _dot1(ah, bh, dims) + _dot1(ah, bl, dims)


def _group_indicator(n, g):
    row = jax.lax.broadcasted_iota(jnp.int32, (n, n // g), 0) // g
    col = jax.lax.broadcasted_iota(jnp.int32, (n, n // g), 1)
    return (row == col).astype(jnp.float32)


def _main_kernel(emb_ref, vw_ref, vb_ref, x_ref, gw_ref, gb_ref, uw_ref,
                 ub_ref, u_ref, sel_ref, logp_ref, aux_ref,
                 veth_ref, vetl_ref, uwh_ref, uwl_ref, accp_ref, accm_ref):
    i = pl.program_id(0)
    nblk = pl.num_programs(0)
    g512 = _group_indicator(NR * H, H)

    @pl.when(i == 0)
    def _():
        vwh, vwl = _split(vw_ref[...])
        emh, eml = _split(emb_ref[...])
        vet = _dot3s(vwh, vwl, emh, eml, ((1,), (1,))) + vb_ref[...]
        n2 = _dot2i_l(g512, vet * vet, ((0,), (0,)))
        inv = 1.0 / jnp.maximum(jnp.sqrt(n2), 1e-12)
        vet = vet * _dot2i_l(g512, inv, ((1,), (0,)))
        vh, vl = _split(vet)
        veth_ref[...] = vh
        vetl_ref[...] = vl
        uh, ul = _split(uw_ref[...])
        uwh_ref[...] = uh
        uwl_ref[...] = ul
        accp_ref[...] = jnp.zeros((1, NR), jnp.float32)
        accm_ref[...] = jnp.zeros((1, NR), jnp.float32)

    SB = TB // 4
    iota8 = jax.lax.broadcasted_iota(jnp.int32, (SB, NR), 1)
    grp = jax.lax.broadcasted_iota(jnp.int32, (SB, NR * H), 1) // H
    cind = _group_indicator(R, CW)
    tri8r = jax.lax.broadcasted_iota(jnp.int32, (NCHUNK, NCHUNK), 0)
    tri8c = jax.lax.broadcasted_iota(jnp.int32, (NCHUNK, NCHUNK), 1)
    tri8 = (tri8r <= tri8c).astype(jnp.float32)
    iotc = jax.lax.broadcasted_iota(jnp.int32, (SB, NCHUNK), 1)
    trir = jax.lax.broadcasted_iota(jnp.int32, (CW, CW), 0)
    tric = jax.lax.broadcasted_iota(jnp.int32, (CW, CW), 1)
    tri128 = (trir <= tric).astype(jnp.float32)
    iotl = jax.lax.broadcasted_iota(jnp.int32, (SB, CW), 1)
    gwh, gwl = _split(gw_ref[...])

    psum_t = jnp.zeros((1, NR), jnp.float32)
    msum_t = jnp.zeros((1, NR), jnp.float32)
    for hf in range(4):
        rows = pl.ds(hf * SB, SB)
        xh, xl = _split(x_ref[rows, :])
        logits = _dot3s(xh, xl, gwh, gwl, ((1,), (1,))) + gb_ref[...]
        v1 = jnp.max(logits, axis=1, keepdims=True)
        i1 = jnp.min(jnp.where(logits == v1, iota8, NR), axis=1,
                     keepdims=True)
        masked = jnp.where(iota8 == i1, -jnp.inf, logits)
        v2 = jnp.max(masked, axis=1, keepdims=True)
        i2 = jnp.min(jnp.where(masked == v2, iota8, NR), axis=1,
                     keepdims=True)
        e21 = jnp.exp(v2 - v1)
        g0 = 1.0 / (1.0 + e21)
        g1 = e21 * g0
        pe = jnp.exp(logits - v1)
        probs = pe / jnp.sum(pe, axis=1, keepdims=True)
        psum_t = psum_t + jnp.sum(probs, axis=0, keepdims=True)
        msum_t = msum_t + jnp.sum(
            (iota8 == i1).astype(jnp.float32)
            + (iota8 == i2).astype(jnp.float32), axis=0, keepdims=True)
        ux = (_dot3s(xh, xl, uwh_ref[...], uwl_ref[...], ((1,), (1,)))
              + ub_ref[...])
        n2 = _dot2i(ux * ux, g512, ((1,), (0,)))
        inv = 1.0 / jnp.maximum(jnp.sqrt(n2), 1e-12)
        ux = ux * _dot2i(inv, g512, ((1,), (1,)))
        uxh, uxl = _split(ux)
        z0h = jnp.where(grp == i1, uxh, 0)
        z0l = jnp.where(grp == i1, uxl, 0)
        z1h = jnp.where(grp == i2, uxh, 0)
        z1l = jnp.where(grp == i2, uxl, 0)
        s0 = _dot3s(z0h, z0l, veth_ref[...], vetl_ref[...], ((1,), (0,)))
        s1 = _dot3s(z1h, z1l, veth_ref[...], vetl_ref[...], ((1,), (0,)))
        e0 = jnp.exp(s0)
        e1 = jnp.exp(s1)
        a0 = g0 / jnp.sum(e0, axis=1, keepdims=True)
        a1 = g1 / jnp.sum(e1, axis=1, keepdims=True)
        rea = a0 * e0 + a1 * e1
        u = u_ref[rows, :]
        csum = _dot2i(rea, cind, ((1,), (0,)))
        ccs = _dot2i(csum, tri8, ((1,), (0,)))
        crossed = ccs > u
        cstar = jnp.min(jnp.where(crossed, iotc, NCHUNK), axis=1,
                        keepdims=True)
        found = cstar < NCHUNK
        prev = ccs - csum
        prevsel = jnp.sum(jnp.where(iotc == cstar, prev, 0.0), axis=1,
                          keepdims=True)
        chunk = jnp.zeros((SB, CW), jnp.float32)
        for c in range(NCHUNK):
            chunk = chunk + jnp.where(cstar == c,
                                      rea[:, c * CW:(c + 1) * CW], 0.0)
        wcs = _dot2i(chunk, tri128, ((1,), (0,))) + prevsel
        lmin = jnp.min(jnp.where(wcs > u, iotl, CW), axis=1, keepdims=True)
        lsel = jnp.where(lmin >= CW, CW - 1, lmin)
        selected = jnp.where(found, cstar * CW + lsel, 0)
        pick = jnp.sum(jnp.where(iotl == lsel, chunk, 0.0), axis=1,
                       keepdims=True)
        pick = jnp.where(found, pick, rea[:, 0:1])
        sel_ref[rows, :] = selected
        logp_ref[rows, :] = jnp.log(pick)

    accp_ref[...] += psum_t
    accm_ref[...] += msum_t

    @pl.when(i == nblk - 1)
    def _():
        ep = accp_ref[...] / B
        em = accm_ref[...] / B
        aux_ref[...] = NR * jnp.sum(ep * em, axis=1, keepdims=True) * AUX


@jax.jit
def kernel(x, reasoning_embeddings, Gw, Gb, Uw, Ub, Vw, Vb):
    vw_flat = Vw.reshape(NR * H, D)
    vb_col = Vb.reshape(NR * H, 1)
    uw_flat = Uw.reshape(NR * H, D)
    ub_row = Ub.reshape(1, NR * H)
    gb_row = Gb.reshape(1, NR)
    rnd = jnp.asarray(_RND)

    nblk = B // TB
    blk = lambda i: (i, 0)
    const = lambda i: (0, 0)
    sel, logp, aux = pl.pallas_call(
        _main_kernel,
        grid=(nblk,),
        in_specs=[
            pl.BlockSpec((R, D), const),
            pl.BlockSpec((NR * H, D), const),
            pl.BlockSpec((NR * H, 1), const),
            pl.BlockSpec((TB, D), blk),
            pl.BlockSpec((NR, D), const),
            pl.BlockSpec((1, NR), const),
            pl.BlockSpec((NR * H, D), const),
            pl.BlockSpec((1, NR * H), const),
            pl.BlockSpec((TB, 1), blk),
        ],
        out_specs=[
            pl.BlockSpec((TB, 1), blk),
            pl.BlockSpec((TB, 1), blk),
            pl.BlockSpec((1, 1), const),
        ],
        out_shape=[
            jax.ShapeDtypeStruct((B, 1), jnp.int32),
            jax.ShapeDtypeStruct((B, 1), jnp.float32),
            jax.ShapeDtypeStruct((1, 1), jnp.float32),
        ],
        scratch_shapes=[
            pltpu.VMEM((NR * H, R), jnp.bfloat16),
            pltpu.VMEM((NR * H, R), jnp.bfloat16),
            pltpu.VMEM((NR * H, D), jnp.bfloat16),
            pltpu.VMEM((NR * H, D), jnp.bfloat16),
            pltpu.VMEM((1, NR), jnp.float32),
            pltpu.VMEM((1, NR), jnp.float32),
        ],
    )(reasoning_embeddings, vw_flat, vb_col, x, Gw, gb_row, uw_flat,
      ub_row, rnd)

    return (sel[:, 0], logp, aux[0, 0])


# speedup vs baseline: 1.0174x; 1.0174x over previous
import jax
import jax.numpy as jnp
import numpy as np
from jax.experimental import pallas as pl
from jax.experimental.pallas import tpu as pltpu

B = 4096
D = 384
H = 64
R = 1024
NR = 8
AUX = 0.05
TB = 1024
NCHUNK = 8
CW = R // NCHUNK

_PREC = jax.lax.Precision.HIGHEST

def _threefry_uniform(seed, n):
    old = np.seterr(over="ignore")
    k0, k1 = np.uint32(0), np.uint32(seed)
    ks2 = np.uint32(0x1BD11BDA) ^ k0 ^ k1
    rot = [[13, 15, 26, 6], [17, 29, 16, 24]]
    x0 = np.zeros(n, np.uint32) + k0
    x1 = np.arange(n, dtype=np.uint32) + k1
    ks = [k0, k1, ks2]
    for i in range(5):
        for r in rot[i % 2]:
            x0 = x0 + x1
            x1 = (x1 << np.uint32(r)) | (x1 >> np.uint32(32 - r))
            x1 = x1 ^ x0
        x0 = x0 + ks[(i + 1) % 3]
        x1 = x1 + ks[(i + 2) % 3] + np.uint32(i + 1)
    bits = x0 ^ x1
    np.seterr(**old)
    bits = (bits >> np.uint32(9)) | np.uint32(0x3F800000)
    return bits.view(np.float32) - np.float32(1.0)


_RND = _threefry_uniform(42, B).reshape(B, 1)


def _dot(a, b, dims):
    return jax.lax.dot_general(a, b, (dims, ((), ())),
                               preferred_element_type=jnp.float32,
                               precision=_PREC)


def _split(a):
    hi = a.astype(jnp.bfloat16)
    lo = (a - hi.astype(jnp.float32)).astype(jnp.bfloat16)
    return hi, lo


def _dot1(a, b, dims):
    return jax.lax.dot_general(a, b, (dims, ((), ())),
                               preferred_element_type=jnp.float32,
                               precision=jax.lax.Precision.DEFAULT)


def _dot3s(a_hi, a_lo, b_hi, b_lo, dims):
    return (_dot1(a_hi, b_hi, dims) + _dot1(a_lo, b_hi, dims)
            + _dot1(a_hi, b_lo, dims))


def _dot2i(a, b_exact, dims):
    ah, al = _split(a)
    bh = b_exact.astype(jnp.bfloat16)
    return _dot1(ah, bh, dims) + _dot1(al, bh, dims)


def _dot2i_l(a_exact, b, dims):
    bh, bl = _split(b)
    ah = a_exact.astype(jnp.bfloat16)
    return _dot1(ah, bh, dims) + _dot1(ah, bl, dims)


def _group_indicator(n, g):
    row = jax.lax.broadcasted_iota(jnp.int32, (n, n // g), 0) // g
    col = jax.lax.broadcasted_iota(jnp.int32, (n, n // g), 1)
    return (row == col).astype(jnp.float32)


def _main_kernel(emb_ref, vw_ref, vb_ref, x_ref, gw_ref, gb_ref, uw_ref,
                 ub_ref, u_ref, sel_ref, logp_ref, aux_ref,
                 veth_ref, vetl_ref, uwh_ref, uwl_ref, accp_ref, accm_ref):
    i = pl.program_id(0)
    nblk = pl.num_programs(0)
    g512 = _group_indicator(NR * H, H)

    @pl.when(i == 0)
    def _():
        vwh, vwl = _split(vw_ref[...])
        emh, eml = _split(emb_ref[...])
        vet = _dot3s(vwh, vwl, emh, eml, ((1,), (1,))) + vb_ref[...]
        n2 = _dot2i_l(g512, vet * vet, ((0,), (0,)))
        inv = 1.0 / jnp.maximum(jnp.sqrt(n2), 1e-12)
        vet = vet * _dot2i_l(g512, inv, ((1,), (0,)))
        vh, vl = _split(vet)
        veth_ref[...] = vh
        vetl_ref[...] = vl
        uh, ul = _split(uw_ref[...])
        uwh_ref[...] = uh
        uwl_ref[...] = ul
        accp_ref[...] = jnp.zeros((1, NR), jnp.float32)
        accm_ref[...] = jnp.zeros((1, NR), jnp.float32)

    SB = TB // 2
    iota8 = jax.lax.broadcasted_iota(jnp.int32, (SB, NR), 1)
    grp = jax.lax.broadcasted_iota(jnp.int32, (SB, NR * H), 1) // H
    cind = _group_indicator(R, CW)
    tri8r = jax.lax.broadcasted_iota(jnp.int32, (NCHUNK, NCHUNK), 0)
    tri8c = jax.lax.broadcasted_iota(jnp.int32, (NCHUNK, NCHUNK), 1)
    tri8 = (tri8r <= tri8c).astype(jnp.float32)
    iotc = jax.lax.broadcasted_iota(jnp.int32, (SB, NCHUNK), 1)
    trir = jax.lax.broadcasted_iota(jnp.int32, (CW, CW), 0)
    tric = jax.lax.broadcasted_iota(jnp.int32, (CW, CW), 1)
    tri128 = (trir <= tric).astype(jnp.float32)
    iotl = jax.lax.broadcasted_iota(jnp.int32, (SB, CW), 1)
    gwh, gwl = _split(gw_ref[...])

    psum_t = jnp.zeros((1, NR), jnp.float32)
    msum_t = jnp.zeros((1, NR), jnp.float32)
    for hf in range(2):
        rows = pl.ds(hf * SB, SB)
        xh, xl = _split(x_ref[rows, :])
        logits = _dot3s(xh, xl, gwh, gwl, ((1,), (1,))) + gb_ref[...]
        v1 = jnp.max(logits, axis=1, keepdims=True)
        i1 = jnp.min(jnp.where(logits == v1, iota8, NR), axis=1,
                     keepdims=True)
        masked = jnp.where(iota8 == i1, -jnp.inf, logits)
        v2 = jnp.max(masked, axis=1, keepdims=True)
        i2 = jnp.min(jnp.where(masked == v2, iota8, NR), axis=1,
                     keepdims=True)
        e21 = jnp.exp(v2 - v1)
        g0 = 1.0 / (1.0 + e21)
        g1 = e21 * g0
        pe = jnp.exp(logits - v1)
        probs = pe / jnp.sum(pe, axis=1, keepdims=True)
        psum_t = psum_t + jnp.sum(probs, axis=0, keepdims=True)
        msum_t = msum_t + jnp.sum(
            (iota8 == i1).astype(jnp.float32)
            + (iota8 == i2).astype(jnp.float32), axis=0, keepdims=True)
        ux = (_dot3s(xh, xl, uwh_ref[...], uwl_ref[...], ((1,), (1,)))
              + ub_ref[...])
        n2 = _dot2i(ux * ux, g512, ((1,), (0,)))
        inv = 1.0 / jnp.maximum(jnp.sqrt(n2), 1e-12)
        ux = ux * _dot2i(inv, g512, ((1,), (1,)))
        uxh, uxl = _split(ux)
        z0h = jnp.where(grp == i1, uxh, 0)
        z0l = jnp.where(grp == i1, uxl, 0)
        z1h = jnp.where(grp == i2, uxh, 0)
        z1l = jnp.where(grp == i2, uxl, 0)
        s0 = _dot3s(z0h, z0l, veth_ref[...], vetl_ref[...], ((1,), (0,)))
        s1 = _dot3s(z1h, z1l, veth_ref[...], vetl_ref[...], ((1,), (0,)))
        e0 = jnp.exp(s0)
        e1 = jnp.exp(s1)
        a0 = g0 / jnp.sum(e0, axis=1, keepdims=True)
        a1 = g1 / jnp.sum(e1, axis=1, keepdims=True)
        rea = a0 * e0 + a1 * e1
        u = u_ref[rows, :]
        csum = _dot2i(rea, cind, ((1,), (0,)))
        ccs = _dot2i(csum, tri8, ((1,), (0,)))
        crossed = ccs > u
        cstar = jnp.min(jnp.where(crossed, iotc, NCHUNK), axis=1,
                        keepdims=True)
        found = cstar < NCHUNK
        prev = ccs - csum
        prevsel = jnp.sum(jnp.where(iotc == cstar, prev, 0.0), axis=1,
                          keepdims=True)
        chunk = jnp.zeros((SB, CW), jnp.float32)
        for c in range(NCHUNK):
            chunk = chunk + jnp.where(cstar == c,
                                      rea[:, c * CW:(c + 1) * CW], 0.0)
        wcs = _dot2i(chunk, tri128, ((1,), (0,))) + prevsel
        lmin = jnp.min(jnp.where(wcs > u, iotl, CW), axis=1, keepdims=True)
        lsel = jnp.where(lmin >= CW, CW - 1, lmin)
        selected = jnp.where(found, cstar * CW + lsel, 0)
        pick = jnp.sum(jnp.where(iotl == lsel, chunk, 0.0), axis=1,
                       keepdims=True)
        pick = jnp.where(found, pick, rea[:, 0:1])
        sel_ref[rows, :] = selected
        logp_ref[rows, :] = jnp.log(pick)

    accp_ref[...] += psum_t
    accm_ref[...] += msum_t

    @pl.when(i == nblk - 1)
    def _():
        ep = accp_ref[...] / B
        em = accm_ref[...] / B
        aux_ref[...] = NR * jnp.sum(ep * em, axis=1, keepdims=True) * AUX


@jax.jit
def kernel(x, reasoning_embeddings, Gw, Gb, Uw, Ub, Vw, Vb):
    vw_flat = Vw.reshape(NR * H, D)
    vb_col = Vb.reshape(NR * H, 1)
    uw_flat = Uw.reshape(NR * H, D)
    ub_row = Ub.reshape(1, NR * H)
    gb_row = Gb.reshape(1, NR)
    rnd = jnp.asarray(_RND)

    nblk = B // TB
    blk = lambda i: (i, 0)
    const = lambda i: (0, 0)
    sel, logp, aux = pl.pallas_call(
        _main_kernel,
        grid=(nblk,),
        in_specs=[
            pl.BlockSpec((R, D), const),
            pl.BlockSpec((NR * H, D), const),
            pl.BlockSpec((NR * H, 1), const),
            pl.BlockSpec((TB, D), blk),
            pl.BlockSpec((NR, D), const),
            pl.BlockSpec((1, NR), const),
            pl.BlockSpec((NR * H, D), const),
            pl.BlockSpec((1, NR * H), const),
            pl.BlockSpec((TB, 1), blk),
        ],
        out_specs=[
            pl.BlockSpec((TB, 1), blk),
            pl.BlockSpec((TB, 1), blk),
            pl.BlockSpec((1, 1), const),
        ],
        out_shape=[
            jax.ShapeDtypeStruct((B, 1), jnp.int32),
            jax.ShapeDtypeStruct((B, 1), jnp.float32),
            jax.ShapeDtypeStruct((1, 1), jnp.float32),
        ],
        scratch_shapes=[
            pltpu.VMEM((NR * H, R), jnp.bfloat16),
            pltpu.VMEM((NR * H, R), jnp.bfloat16),
            pltpu.VMEM((NR * H, D), jnp.bfloat16),
            pltpu.VMEM((NR * H, D), jnp.bfloat16),
            pltpu.VMEM((1, NR), jnp.float32),
            pltpu.VMEM((1, NR), jnp.float32),
        ],
    )(reasoning_embeddings, vw_flat, vb_col, x, Gw, gb_row, uw_flat,
      ub_row, rnd)

    return (sel[:, 0], logp, aux[0, 0])
